# Initial kernel scaffold; baseline (speedup 1.0000x reference)
import functools

import jax
import jax.numpy as jnp
from jax import lax
from jax.experimental import pallas as pl
from jax.experimental.pallas import tpu as pltpu
from jax.experimental.pallas import tpu_sc as plsc

N = 100000
E = 1600000
NDF = 38
NH = 32
NE = 32
D = 64
HID = 64

NC = 2
NS = 16
F = 16

ROWS_PER_TILE = N // NS
ZCHUNK = 625
SEG_CHUNK = 1000
HIST_CHUNK = 1000

_MESH = plsc.VectorSubcoreMesh(
    core_axis_name="c", subcore_axis_name="s", num_cores=NC, num_subcores=NS)


def _fill_rows(ref, nrows, vec):
    @pl.loop(0, nrows)
    def _(i):
        ref[i] = vec


def _zero_acc(acc_sh, zeros_v, s):
    base = s * ROWS_PER_TILE

    @pl.loop(0, ROWS_PER_TILE // ZCHUNK)
    def _(j):
        pltpu.sync_copy(zeros_v, acc_sh.at[pl.ds(base + j * ZCHUNK, ZCHUNK)])


@functools.partial(
    pl.kernel,
    out_type=jax.ShapeDtypeStruct((NC, N, F), jnp.float32),
    mesh=_MESH,
    scratch_types=[
        pltpu.VMEM((HIST_CHUNK,), jnp.int32),
        pltpu.VMEM((HIST_CHUNK, F), jnp.float32),
        pltpu.VMEM((ZCHUNK, F), jnp.float32),
        pltpu.VMEM_SHARED((N, F), jnp.float32),
    ],
)
def _sc_hist(dst_hbm, out_hbm, idx_v, ones_v, zeros_v, acc_sh):
    c = lax.axis_index("c")
    s = lax.axis_index("s")
    lane = lax.iota(jnp.int32, 16)
    _fill_rows(zeros_v, ZCHUNK, jnp.zeros((16,), jnp.float32))
    _fill_rows(ones_v, HIST_CHUNK,
               jnp.where(lane == 0, 1.0, 0.0).astype(jnp.float32))
    _zero_acc(acc_sh, zeros_v, s)
    plsc.subcore_barrier()

    edges_per_tile = E // (NC * NS)
    tile_base = (c * NS + s) * edges_per_tile

    @pl.loop(0, edges_per_tile // HIST_CHUNK)
    def _(k):
        off = pl.multiple_of(tile_base + k * HIST_CHUNK, 8)
        pltpu.sync_copy(dst_hbm.at[pl.ds(off, HIST_CHUNK)], idx_v)
        pltpu.sync_copy(ones_v, acc_sh.at[idx_v], add=True)

    plsc.subcore_barrier()
    row = s * ROWS_PER_TILE
    pltpu.sync_copy(acc_sh.at[pl.ds(row, ROWS_PER_TILE)],
                    out_hbm.at[c].at[pl.ds(row, ROWS_PER_TILE)])


@functools.partial(
    pl.kernel,
    out_type=jax.ShapeDtypeStruct((NC, N, F), jnp.float32),
    mesh=_MESH,
    scratch_types=[
        pltpu.VMEM((SEG_CHUNK,), jnp.int32),
        pltpu.VMEM((SEG_CHUNK,), jnp.int32),
        pltpu.VMEM((SEG_CHUNK, F), jnp.float32),
        pltpu.VMEM((ZCHUNK, F), jnp.float32),
        pltpu.VMEM_SHARED((N, F), jnp.float32),
    ],
)
def _sc_segsum(hsa_hbm, hsb_hbm, src_hbm, dst_hbm, out_hbm,
               src_v, dst_v, rows_v, zeros_v, acc_sh):
    c = lax.axis_index("c")
    s = lax.axis_index("s")
    _fill_rows(zeros_v, ZCHUNK, jnp.zeros((16,), jnp.float32))
    _zero_acc(acc_sh, zeros_v, s)
    plsc.subcore_barrier()

    edges_per_tile = E // NS
    tile_base = s * edges_per_tile

    def edge_loop(hs_hbm):
        @pl.loop(0, edges_per_tile // SEG_CHUNK)
        def _(k):
            off = pl.multiple_of(tile_base + k * SEG_CHUNK, 8)
            pltpu.sync_copy(src_hbm.at[pl.ds(off, SEG_CHUNK)], src_v)
            pltpu.sync_copy(dst_hbm.at[pl.ds(off, SEG_CHUNK)], dst_v)
            pltpu.sync_copy(hs_hbm.at[src_v], rows_v)
            pltpu.sync_copy(rows_v, acc_sh.at[dst_v], add=True)

    @pl.when(c == 0)
    def _():
        edge_loop(hsa_hbm)

    @pl.when(c == 1)
    def _():
        edge_loop(hsb_hbm)

    plsc.subcore_barrier()
    row = s * ROWS_PER_TILE
    pltpu.sync_copy(acc_sh.at[pl.ds(row, ROWS_PER_TILE)],
                    out_hbm.at[c].at[pl.ds(row, ROWS_PER_TILE)])


NB = 10000


def _lrelu(x):
    return jnp.where(x >= 0, x, 0.01 * x)


def _dot(a, b):
    return jnp.dot(a, b, preferred_element_type=jnp.float32)


def _tc_a_body(dx, h0, h1, wg0, bg0, wg1, hs1a, hs1b, dinv_o):
    x_d = dx[:, 6:NDF]
    xg0 = _lrelu(_dot(x_d, wg0[...]) + bg0[...])
    deg = 1.0 + h0[:, 0:1] + h1[:, 0:1]
    dinv = lax.rsqrt(deg)
    hs1 = _dot(xg0, wg1[...]) * dinv
    hs1a[...] = hs1[:, :F]
    hs1b[...] = hs1[:, F:]
    dinv_o[...] = dinv


def _tc_b_body(sa, sb, hs1a, hs1b, dinv, bg1, wg2, hs2a, hs2b):
    s1 = jnp.concatenate([sa[...], sb[...]], axis=1)
    hs1 = jnp.concatenate([hs1a[...], hs1b[...]], axis=1)
    xg1 = _lrelu(dinv[...] * (s1 + hs1) + bg1[...])
    hs2 = _dot(xg1, wg2[...]) * dinv[...]
    hs2a[...] = hs2[:, :F]
    hs2b[...] = hs2[:, F:]


def _tc_c_body(dx, sa, sb, hs2a, hs2b, dinv, bg2, wd, bd, cwt, cb,
               wp1, bp1, wp2, bp2, y_o):
    s2 = jnp.concatenate([sa[...], sb[...]], axis=1)
    hs2 = jnp.concatenate([hs2a[...], hs2b[...]], axis=1)
    xg2 = _lrelu(dinv[...] * (s2 + hs2) + bg2[...])
    x = jnp.concatenate([dx[:, 6:NDF], xg2], axis=1)
    h = x
    for i in range(3):
        h = h + _lrelu(_dot(h, wd[i]) + bd[i])
    x0 = x
    xl = x
    for i in range(2):
        xl = x0 * _dot(xl, cwt[:, i:i + 1]) + cb[i] + xl
    z = jnp.concatenate([h, xl], axis=1)
    p = _lrelu(_dot(z, wp1[...]) + bp1[...])
    y_o[...] = jax.nn.sigmoid(_dot(p, wp2[...]) + bp2[...])


def _row_spec(w):
    return pl.BlockSpec((NB, w), lambda i: (i, 0))


def _full_spec(shape):
    nd = len(shape)
    return pl.BlockSpec(shape, lambda i, _n=nd: (0,) * _n)


def kernel(discrete_x, continous_x, edge_index, edge_attr, churn_date,
           W_g0, b_g0, W_g1, b_g1, W_g2, b_g2, Wd, bd, cw, cb,
           Wp1, bp1, Wp2, bp2):
    f32 = jnp.float32
    src = edge_index[0]
    dst = edge_index[1]

    hist = _sc_hist(dst)

    grid = (N // NB,)
    hs1a, hs1b, dinv = pl.pallas_call(
        _tc_a_body,
        grid=grid,
        in_specs=[_row_spec(NDF), _row_spec(F), _row_spec(F),
                  _full_spec((NH, NE)), _full_spec((1, NE)),
                  _full_spec((NE, NE))],
        out_specs=[_row_spec(F), _row_spec(F), _row_spec(1)],
        out_shape=[jax.ShapeDtypeStruct((N, F), f32),
                   jax.ShapeDtypeStruct((N, F), f32),
                   jax.ShapeDtypeStruct((N, 1), f32)],
    )(discrete_x, hist[0], hist[1], W_g0, b_g0.reshape(1, NE), W_g1)

    s1 = _sc_segsum(hs1a, hs1b, src, dst)

    hs2a, hs2b = pl.pallas_call(
        _tc_b_body,
        grid=grid,
        in_specs=[_row_spec(F), _row_spec(F), _row_spec(F), _row_spec(F),
                  _row_spec(1), _full_spec((1, NE)), _full_spec((NE, NE))],
        out_specs=[_row_spec(F), _row_spec(F)],
        out_shape=[jax.ShapeDtypeStruct((N, F), f32),
                   jax.ShapeDtypeStruct((N, F), f32)],
    )(s1[0], s1[1], hs1a, hs1b, dinv, b_g1.reshape(1, NE), W_g2)

    s2 = _sc_segsum(hs2a, hs2b, src, dst)

    y = pl.pallas_call(
        _tc_c_body,
        grid=grid,
        in_specs=[_row_spec(NDF), _row_spec(F), _row_spec(F),
                  _row_spec(F), _row_spec(F), _row_spec(1),
                  _full_spec((1, NE)),
                  _full_spec((3, D, D)), _full_spec((3, 1, D)),
                  _full_spec((D, 2)), _full_spec((2, 1, D)),
                  _full_spec((2 * D, HID)), _full_spec((1, HID)),
                  _full_spec((HID, 1)), _full_spec((1, 1))],
        out_specs=[_row_spec(1)],
        out_shape=[jax.ShapeDtypeStruct((N, 1), f32)],
    )(discrete_x, s2[0], s2[1], hs2a, hs2b, dinv, b_g2.reshape(1, NE),
      Wd, bd.reshape(3, 1, D), cw.T, cb.reshape(2, 1, D),
      Wp1, bp1.reshape(1, HID), Wp2, bp2.reshape(1, 1))

    return y[0].reshape(N)


# speedup vs baseline: 20.7071x; 20.7071x over previous
import functools

import jax
import jax.numpy as jnp
from jax import lax
from jax.experimental import pallas as pl
from jax.experimental.pallas import tpu as pltpu
from jax.experimental.pallas import tpu_sc as plsc

N = 100000
E = 1600000
NDF = 38
NH = 32
NE = 32
D = 64
HID = 64

NC = 2
NS = 16
F = 16

N_ACC = N
ROWS_PER_TILE = N_ACC // NS
ZCHUNK = 625
SEG_CHUNK = 1000
HIST_CHUNK = 1000

_MESH = plsc.VectorSubcoreMesh(
    core_axis_name="c", subcore_axis_name="s", num_cores=NC, num_subcores=NS)


def _fill_rows(ref, nrows, vec):
    @pl.loop(0, nrows)
    def _(i):
        ref[i] = vec


def _zero_acc(acc_sh, zeros_v, s):
    base = s * ROWS_PER_TILE

    @pl.loop(0, ROWS_PER_TILE // ZCHUNK)
    def _(j):
        pltpu.sync_copy(zeros_v, acc_sh.at[pl.ds(base + j * ZCHUNK, ZCHUNK)])


@functools.partial(
    pl.kernel,
    out_type=jax.ShapeDtypeStruct((NC, N_ACC, F), jnp.float32),
    mesh=_MESH,
    compiler_params=pltpu.CompilerParams(use_tc_tiling_on_sc=False),
    scratch_types=[
        pltpu.VMEM((SEG_CHUNK,), jnp.int32),
        pltpu.VMEM((SEG_CHUNK,), jnp.int32),
        pltpu.VMEM((SEG_CHUNK, F), jnp.float32),
        pltpu.VMEM((ZCHUNK, F), jnp.float32),
        pltpu.VMEM_SHARED((N_ACC, F), jnp.float32),
    ],
)
def _sc_segsum(hsa_hbm, hsb_hbm, src_hbm, dst_hbm, out_hbm,
               src_v, dst_v, rows_v, zeros_v, acc_sh):
    c = lax.axis_index("c")
    s = lax.axis_index("s")
    _fill_rows(zeros_v, ZCHUNK, jnp.zeros((16,), jnp.float32))
    _zero_acc(acc_sh, zeros_v, s)
    plsc.subcore_barrier()

    edges_per_tile = E // NS
    tile_base = s * edges_per_tile

    def edge_loop(hs_hbm):
        @pl.loop(0, edges_per_tile // SEG_CHUNK)
        def _(k):
            off = pl.multiple_of(tile_base + k * SEG_CHUNK, 8)
            pltpu.sync_copy(src_hbm.at[pl.ds(off, SEG_CHUNK)], src_v)
            pltpu.sync_copy(dst_hbm.at[pl.ds(off, SEG_CHUNK)], dst_v)
            pltpu.sync_copy(hs_hbm.at[src_v], rows_v)
            pltpu.sync_copy(rows_v, acc_sh.at[dst_v], add=True)

    @pl.when(c == 0)
    def _():
        edge_loop(hsa_hbm)

    @pl.when(c == 1)
    def _():
        edge_loop(hsb_hbm)

    plsc.subcore_barrier()
    row = s * ROWS_PER_TILE
    pltpu.sync_copy(acc_sh.at[pl.ds(row, ROWS_PER_TILE)],
                    out_hbm.at[c].at[pl.ds(row, ROWS_PER_TILE)])


NB = 2000


def _lrelu(x):
    return jnp.where(x >= 0, x, 0.01 * x)


def _dot(a, b):
    return jnp.dot(a, b, preferred_element_type=jnp.float32)


def _tc_a_body(dx, h0, wg0, bg0, wg1, hs1a, hs1b, dinv_o):
    x_d = dx[:, 6:NDF]
    xg0 = _lrelu(_dot(x_d, wg0[...]) + bg0[...])
    deg = 1.0 + h0[:, 0:1]
    dinv = lax.rsqrt(deg)
    hs1 = _dot(xg0, wg1[...]) * dinv
    hs1a[...] = hs1[:, :F]
    hs1b[...] = hs1[:, F:]
    dinv_o[...] = dinv


def _tc_b_body(sa, sb, hs1a, hs1b, dinv, bg1, wg2, hs2a, hs2b):
    s1 = jnp.concatenate([sa[...], sb[...]], axis=1)
    hs1 = jnp.concatenate([hs1a[...], hs1b[...]], axis=1)
    xg1 = _lrelu(dinv[...] * (s1 + hs1) + bg1[...])
    hs2 = _dot(xg1, wg2[...]) * dinv[...]
    hs2a[...] = hs2[:, :F]
    hs2b[...] = hs2[:, F:]


def _tc_c_body(dx, sa, sb, hs2a, hs2b, dinv, bg2, wd, bd, cwt, cb,
               wp1, bp1, wp2, bp2, y_o):
    s2 = jnp.concatenate([sa[...], sb[...]], axis=1)
    hs2 = jnp.concatenate([hs2a[...], hs2b[...]], axis=1)
    xg2 = _lrelu(dinv[...] * (s2 + hs2) + bg2[...])
    x = jnp.concatenate([dx[:, 6:NDF], xg2], axis=1)
    h = x
    for i in range(3):
        h = h + _lrelu(_dot(h, wd[i]) + bd[i])
    x0 = x
    xl = x
    for i in range(2):
        xl = x0 * _dot(xl, cwt[:, i:i + 1]) + cb[i] + xl
    z = jnp.concatenate([h, xl], axis=1)
    p = _lrelu(_dot(z, wp1[...]) + bp1[...])
    y_o[...] = jax.nn.sigmoid(_dot(p, wp2[...]) + bp2[...])


def _row_spec(w):
    return pl.BlockSpec((NB, w), lambda i: (i, 0))


def _full_spec(shape):
    nd = len(shape)
    return pl.BlockSpec(shape, lambda i, _n=nd: (0,) * _n)


def kernel(discrete_x, continous_x, edge_index, edge_attr, churn_date,
           W_g0, b_g0, W_g1, b_g1, W_g2, b_g2, Wd, bd, cw, cb,
           Wp1, bp1, Wp2, bp2):
    f32 = jnp.float32
    src = edge_index[0]
    dst = edge_index[1]

    ones_tab = jnp.ones((N, F), jnp.float32)
    hist = _sc_segsum(ones_tab, ones_tab, src, dst)

    grid = (N // NB,)
    hs1a, hs1b, dinv = pl.pallas_call(
        _tc_a_body,
        grid=grid,
        in_specs=[_row_spec(NDF), _row_spec(F),
                  _full_spec((NH, NE)), _full_spec((1, NE)),
                  _full_spec((NE, NE))],
        out_specs=[_row_spec(F), _row_spec(F), _row_spec(1)],
        out_shape=[jax.ShapeDtypeStruct((N, F), f32),
                   jax.ShapeDtypeStruct((N, F), f32),
                   jax.ShapeDtypeStruct((N, 1), f32)],
    )(discrete_x, hist[0, :N], W_g0, b_g0.reshape(1, NE), W_g1)

    s1 = _sc_segsum(hs1a, hs1b, src, dst)

    hs2a, hs2b = pl.pallas_call(
        _tc_b_body,
        grid=grid,
        in_specs=[_row_spec(F), _row_spec(F), _row_spec(F), _row_spec(F),
                  _row_spec(1), _full_spec((1, NE)), _full_spec((NE, NE))],
        out_specs=[_row_spec(F), _row_spec(F)],
        out_shape=[jax.ShapeDtypeStruct((N, F), f32),
                   jax.ShapeDtypeStruct((N, F), f32)],
    )(s1[0, :N], s1[1, :N], hs1a, hs1b, dinv, b_g1.reshape(1, NE), W_g2)

    s2 = _sc_segsum(hs2a, hs2b, src, dst)

    y = pl.pallas_call(
        _tc_c_body,
        grid=grid,
        in_specs=[_row_spec(NDF), _row_spec(F), _row_spec(F),
                  _row_spec(F), _row_spec(F), _row_spec(1),
                  _full_spec((1, NE)),
                  _full_spec((3, D, D)), _full_spec((3, 1, D)),
                  _full_spec((D, 2)), _full_spec((2, 1, D)),
                  _full_spec((2 * D, HID)), _full_spec((1, HID)),
                  _full_spec((HID, 1)), _full_spec((1, 1))],
        out_specs=[_row_spec(1)],
        out_shape=[jax.ShapeDtypeStruct((N, 1), f32)],
    )(discrete_x, s2[0, :N], s2[1, :N], hs2a, hs2b, dinv, b_g2.reshape(1, NE),
      Wd, bd.reshape(3, 1, D), cw.T, cb.reshape(2, 1, D),
      Wp1, bp1.reshape(1, HID), Wp2, bp2.reshape(1, 1))

    return y[0].reshape(N)


# speedup vs baseline: 29.2108x; 1.4107x over previous
import functools

import jax
import jax.numpy as jnp
from jax import lax
from jax.experimental import pallas as pl
from jax.experimental.pallas import tpu as pltpu
from jax.experimental.pallas import tpu_sc as plsc

N = 100000
E = 1600000
NDF = 38
NH = 32
NE = 32
D = 64
HID = 64

NC = 2
NS = 16
F = 16

N_ACC = N
ROWS_PER_TILE = N_ACC // NS
ZCHUNK = 125
SEG_CHUNK = 800
NCHUNK = (E // NS) // SEG_CHUNK

_MESH = plsc.VectorSubcoreMesh(
    core_axis_name="c", subcore_axis_name="s", num_cores=NC, num_subcores=NS)


def _fill_rows(ref, nrows, vec):
    @pl.loop(0, nrows)
    def _(i):
        ref[i] = vec


def _zero_acc(acc_sh, zeros_v, s):
    base = s * ROWS_PER_TILE

    @pl.loop(0, ROWS_PER_TILE // ZCHUNK)
    def _(j):
        pltpu.sync_copy(zeros_v, acc_sh.at[pl.ds(base + j * ZCHUNK, ZCHUNK)])


@functools.partial(
    pl.kernel,
    out_type=jax.ShapeDtypeStruct((NC, N_ACC, F), jnp.float32),
    mesh=_MESH,
    compiler_params=pltpu.CompilerParams(use_tc_tiling_on_sc=False),
    scratch_types=[
        pltpu.VMEM((SEG_CHUNK,), jnp.int32),
        pltpu.VMEM((SEG_CHUNK,), jnp.int32),
        pltpu.VMEM((SEG_CHUNK,), jnp.int32),
        pltpu.VMEM((SEG_CHUNK,), jnp.int32),
        pltpu.VMEM((SEG_CHUNK, F), jnp.float32),
        pltpu.VMEM((SEG_CHUNK, F), jnp.float32),
        pltpu.VMEM((ZCHUNK, F), jnp.float32),
        pltpu.VMEM_SHARED((N_ACC, F), jnp.float32),
        pltpu.SemaphoreType.DMA,
        pltpu.SemaphoreType.DMA,
        pltpu.SemaphoreType.DMA,
        pltpu.SemaphoreType.DMA,
    ],
)
def _sc_segsum(hs_hbm, src_hbm, dst_hbm, out_hbm,
               src0, src1, dst0, dst1, rows0, rows1, zeros_v, acc_sh,
               sem_i0, sem_i1, sem_g0, sem_g1):
    c = lax.axis_index("c")
    s = lax.axis_index("s")
    _fill_rows(zeros_v, ZCHUNK, jnp.zeros((16,), jnp.float32))
    _zero_acc(acc_sh, zeros_v, s)
    plsc.subcore_barrier()

    edges_per_tile = E // NS
    tile_base = s * edges_per_tile
    half = hs_hbm.at[c]
    bufs = ((src0, dst0, rows0, sem_i0, sem_g0),
            (src1, dst1, rows1, sem_i1, sem_g1))

    def idx_start(ck, b):
        s_, d_, _, si, _ = bufs[b]
        off = pl.multiple_of(tile_base + ck * SEG_CHUNK, 8)
        pltpu.async_copy(src_hbm.at[pl.ds(off, SEG_CHUNK)], s_, si)
        pltpu.async_copy(dst_hbm.at[pl.ds(off, SEG_CHUNK)], d_, si)

    def idx_wait(b):
        s_, d_, _, si, _ = bufs[b]
        pltpu.make_async_copy(src_hbm.at[pl.ds(0, SEG_CHUNK)], s_, si).wait()
        pltpu.make_async_copy(dst_hbm.at[pl.ds(0, SEG_CHUNK)], d_, si).wait()

    def gather_start(b):
        s_, _, r_, _, sg = bufs[b]
        pltpu.async_copy(half.at[s_], r_, sg)

    def gather_wait(b):
        s_, _, r_, _, sg = bufs[b]
        pltpu.make_async_copy(half.at[s_], r_, sg).wait()

    def scatter(b):
        _, d_, r_, _, _ = bufs[b]
        pltpu.sync_copy(r_, acc_sh.at[d_], add=True)

    idx_start(0, 0)
    idx_start(1, 1)
    idx_wait(0)
    gather_start(0)

    @pl.loop(0, NCHUNK // 2)
    def _(j):
        for b in (0, 1):
            ck = 2 * j + b
            gather_wait(b)

            @pl.when(ck + 1 < NCHUNK)
            def _():
                idx_wait(1 - b)
                gather_start(1 - b)

            scatter(b)

            @pl.when(ck + 2 < NCHUNK)
            def _():
                idx_start(ck + 2, b)

    if NCHUNK % 2:
        gather_wait(0)
        scatter(0)

    plsc.subcore_barrier()
    row = s * ROWS_PER_TILE
    pltpu.sync_copy(acc_sh.at[pl.ds(row, ROWS_PER_TILE)],
                    out_hbm.at[c].at[pl.ds(row, ROWS_PER_TILE)])


NB = 5000


def _lrelu(x):
    return jnp.maximum(x, 0.01 * x)


def _dot(a, b):
    return jnp.dot(a, b, preferred_element_type=jnp.float32)


def _tc_a_body(dx, h0, wg0, bg0, wg1, hs1_o, dinv_o):
    x_d = dx[:, 6:NDF]
    xg0 = _lrelu(_dot(x_d, wg0[...]) + bg0[...])
    deg = 1.0 + h0[:, 0:1]
    dinv = lax.rsqrt(deg)
    hs1 = _dot(xg0, wg1[...]) * dinv
    hs1_o[0] = hs1[:, :F]
    hs1_o[1] = hs1[:, F:]
    dinv_o[...] = dinv


def _tc_b_body(s1_r, hs1_r, dinv, bg1, wg2, hs2_o):
    s1 = jnp.concatenate([s1_r[0], s1_r[1]], axis=1)
    hs1 = jnp.concatenate([hs1_r[0], hs1_r[1]], axis=1)
    xg1 = _lrelu(dinv[...] * (s1 + hs1) + bg1[...])
    hs2 = _dot(xg1, wg2[...]) * dinv[...]
    hs2_o[0] = hs2[:, :F]
    hs2_o[1] = hs2[:, F:]


def _tc_c_body(dx, s2_r, hs2_r, dinv, bg2, wd, bd, cwt, cb,
               wp1, bp1, wp2, bp2, y_o):
    s2 = jnp.concatenate([s2_r[0], s2_r[1]], axis=1)
    hs2 = jnp.concatenate([hs2_r[0], hs2_r[1]], axis=1)
    xg2 = _lrelu(dinv[...] * (s2 + hs2) + bg2[...])
    x = jnp.concatenate([dx[:, 6:NDF], xg2], axis=1)
    h = x
    for i in range(3):
        h = h + _lrelu(_dot(h, wd[i]) + bd[i])
    x0 = x
    xl = x
    for i in range(2):
        xl = x0 * _dot(xl, cwt[:, i:i + 1]) + cb[i] + xl
    z = jnp.concatenate([h, xl], axis=1)
    p = _lrelu(_dot(z, wp1[...]) + bp1[...])
    y_o[...] = jax.nn.sigmoid(_dot(p, wp2[...]) + bp2[...])


def _row_spec(w):
    return pl.BlockSpec((NB, w), lambda i: (i, 0))


def _pair_spec():
    return pl.BlockSpec((2, NB, F), lambda i: (0, i, 0))


def _full_spec(shape):
    nd = len(shape)
    return pl.BlockSpec(shape, lambda i, _n=nd: (0,) * _n)


def kernel(discrete_x, continous_x, edge_index, edge_attr, churn_date,
           W_g0, b_g0, W_g1, b_g1, W_g2, b_g2, Wd, bd, cw, cb,
           Wp1, bp1, Wp2, bp2):
    f32 = jnp.float32
    src = edge_index[0]
    dst = edge_index[1]

    ones_tab = jnp.ones((NC, N, F), jnp.float32)
    hist = _sc_segsum(ones_tab, src, dst)

    grid = (N // NB,)
    hs1, dinv = pl.pallas_call(
        _tc_a_body,
        grid=grid,
        in_specs=[_row_spec(NDF), _row_spec(F),
                  _full_spec((NH, NE)), _full_spec((1, NE)),
                  _full_spec((NE, NE))],
        out_specs=[_pair_spec(), _row_spec(1)],
        out_shape=[jax.ShapeDtypeStruct((NC, N, F), f32),
                   jax.ShapeDtypeStruct((N, 1), f32)],
    )(discrete_x, hist[0, :N], W_g0, b_g0.reshape(1, NE), W_g1)

    s1 = _sc_segsum(hs1, src, dst)

    (hs2,) = pl.pallas_call(
        _tc_b_body,
        grid=grid,
        in_specs=[_pair_spec(), _pair_spec(),
                  _row_spec(1), _full_spec((1, NE)), _full_spec((NE, NE))],
        out_specs=[_pair_spec()],
        out_shape=[jax.ShapeDtypeStruct((NC, N, F), f32)],
    )(s1, hs1, dinv, b_g1.reshape(1, NE), W_g2)

    s2 = _sc_segsum(hs2, src, dst)

    y = pl.pallas_call(
        _tc_c_body,
        grid=grid,
        in_specs=[_row_spec(NDF), _pair_spec(), _pair_spec(), _row_spec(1),
                  _full_spec((1, NE)),
                  _full_spec((3, D, D)), _full_spec((3, 1, D)),
                  _full_spec((D, 2)), _full_spec((2, 1, D)),
                  _full_spec((2 * D, HID)), _full_spec((1, HID)),
                  _full_spec((HID, 1)), _full_spec((1, 1))],
        out_specs=[_row_spec(1)],
        out_shape=[jax.ShapeDtypeStruct((N, 1), f32)],
    )(discrete_x, s2, hs2, dinv, b_g2.reshape(1, NE),
      Wd, bd.reshape(3, 1, D), cw.T, cb.reshape(2, 1, D),
      Wp1, bp1.reshape(1, HID), Wp2, bp2.reshape(1, 1))

    return y[0].reshape(N)


# speedup vs baseline: 35.1249x; 1.2025x over previous
import functools

import jax
import jax.numpy as jnp
from jax import lax
from jax.experimental import pallas as pl
from jax.experimental.pallas import tpu as pltpu
from jax.experimental.pallas import tpu_sc as plsc

N = 100000
E = 1600000
NDF = 38
NH = 32
NE = 32
D = 64
HID = 64

NC = 2
NS = 16
F = 16

N_ACC = N
ROWS_PER_TILE = N_ACC // NS
ZCHUNK = 125
SEG_CHUNK = 800
NCHUNK = (E // NS) // SEG_CHUNK

_MESH = plsc.VectorSubcoreMesh(
    core_axis_name="c", subcore_axis_name="s", num_cores=NC, num_subcores=NS)


def _fill_rows(ref, nrows, vec):
    @pl.loop(0, nrows)
    def _(i):
        ref[i] = vec


def _zero_acc(acc_sh, zeros_v, s):
    base = s * ROWS_PER_TILE

    @pl.loop(0, ROWS_PER_TILE // ZCHUNK)
    def _(j):
        pltpu.sync_copy(zeros_v, acc_sh.at[pl.ds(base + j * ZCHUNK, ZCHUNK)])


@functools.partial(
    pl.kernel,
    out_type=jax.ShapeDtypeStruct((NC * N_ACC, F), jnp.float32),
    mesh=_MESH,
    compiler_params=pltpu.CompilerParams(use_tc_tiling_on_sc=False),
    scratch_types=[
        pltpu.VMEM((SEG_CHUNK,), jnp.int32),
        pltpu.VMEM((SEG_CHUNK,), jnp.int32),
        pltpu.VMEM((SEG_CHUNK,), jnp.int32),
        pltpu.VMEM((SEG_CHUNK,), jnp.int32),
        pltpu.VMEM((SEG_CHUNK, F), jnp.float32),
        pltpu.VMEM((SEG_CHUNK, F), jnp.float32),
        pltpu.VMEM((ZCHUNK, F), jnp.float32),
        pltpu.VMEM_SHARED((N_ACC, F), jnp.float32),
        pltpu.SemaphoreType.DMA,
        pltpu.SemaphoreType.DMA,
        pltpu.SemaphoreType.DMA,
        pltpu.SemaphoreType.DMA,
    ],
)
def _sc_segsum(hs_hbm, ei_hbm, out_hbm,
               src0, src1, dst0, dst1, rows0, rows1, zeros_v, acc_sh,
               sem_i0, sem_i1, sem_g0, sem_g1):
    c = lax.axis_index("c")
    s = lax.axis_index("s")
    _fill_rows(zeros_v, ZCHUNK, jnp.zeros((16,), jnp.float32))
    _zero_acc(acc_sh, zeros_v, s)
    plsc.subcore_barrier()

    edges_per_tile = E // NS
    tile_base = s * edges_per_tile
    half = hs_hbm.at[pl.ds(pl.multiple_of(c * N, 8), N)]
    src_hbm = ei_hbm.at[0]
    dst_hbm = ei_hbm.at[1]
    bufs = ((src0, dst0, rows0, sem_i0, sem_g0),
            (src1, dst1, rows1, sem_i1, sem_g1))

    def idx_start(ck, b):
        s_, d_, _, si, _ = bufs[b]
        off = pl.multiple_of(tile_base + ck * SEG_CHUNK, 8)
        pltpu.async_copy(src_hbm.at[pl.ds(off, SEG_CHUNK)], s_, si)
        pltpu.async_copy(dst_hbm.at[pl.ds(off, SEG_CHUNK)], d_, si)

    def idx_wait(b):
        s_, d_, _, si, _ = bufs[b]
        pltpu.make_async_copy(src_hbm.at[pl.ds(0, SEG_CHUNK)], s_, si).wait()
        pltpu.make_async_copy(dst_hbm.at[pl.ds(0, SEG_CHUNK)], d_, si).wait()

    def gather_start(b):
        s_, _, r_, _, sg = bufs[b]
        pltpu.async_copy(half.at[s_], r_, sg)

    def gather_wait(b):
        s_, _, r_, _, sg = bufs[b]
        pltpu.make_async_copy(half.at[s_], r_, sg).wait()

    def scatter(b):
        _, d_, r_, _, _ = bufs[b]
        pltpu.sync_copy(r_, acc_sh.at[d_], add=True)

    idx_start(0, 0)
    idx_start(1, 1)
    idx_wait(0)
    gather_start(0)

    @pl.loop(0, NCHUNK // 2)
    def _(j):
        for b in (0, 1):
            ck = 2 * j + b
            gather_wait(b)

            @pl.when(ck + 1 < NCHUNK)
            def _():
                idx_wait(1 - b)
                gather_start(1 - b)

            scatter(b)

            @pl.when(ck + 2 < NCHUNK)
            def _():
                idx_start(ck + 2, b)

    if NCHUNK % 2:
        gather_wait(0)
        scatter(0)

    plsc.subcore_barrier()
    row = s * ROWS_PER_TILE
    pltpu.sync_copy(acc_sh.at[pl.ds(row, ROWS_PER_TILE)],
                    out_hbm.at[pl.ds(c * N_ACC + row, ROWS_PER_TILE)])


NB = 5000
NBF = NB // 8
G = N // NB


def _lrelu(x):
    return jnp.maximum(x, 0.01 * x)


def _dot(a, b):
    return jnp.dot(a, b, preferred_element_type=jnp.float32)


def _unpack(flat):
    parts = [flat[:, 16 * k:16 * (k + 1)] for k in range(8)]
    return jnp.stack(parts, axis=1).reshape(NB, F)


def _pack(half):
    h3 = half.reshape(NBF, 8, F)
    return jnp.concatenate([h3[:, k] for k in range(8)], axis=1)


def _dinv_flat(hist):
    return lax.rsqrt(1.0 + hist[0, 0])


def _tc_a_body(dx, hist, wg0, bg0, wg1, hs1_o):
    x_d = dx[:, 6:NDF]
    xg0 = _lrelu(_dot(x_d, wg0[...]) + bg0[...])
    hs1 = _dot(xg0, wg1[...])
    dinvb = _dinv_flat(hist)
    hs1_o[0, 0] = _pack(hs1[:, :F]) * dinvb
    hs1_o[1, 0] = _pack(hs1[:, F:]) * dinvb


def _tc_b_body(s1_r, hs1_r, hist, bg1f, w2bd, hs2_o):
    dinvb = _dinv_flat(hist)
    xg1f = _lrelu(dinvb * (s1_r[:, 0] + hs1_r[:, 0]) + bg1f[...])
    for co in range(2):
        acc = _dot(xg1f[0], w2bd[0, co]) + _dot(xg1f[1], w2bd[1, co])
        hs2_o[co, 0] = acc * dinvb


def _tc_c_body(dx, s2_r, hs2_r, hist, bg2f, wd, bd, cwt, cb,
               wp1, bp1, wp2, bp2, y_o):
    xg2f = _lrelu(_dinv_flat(hist) * (s2_r[:, 0] + hs2_r[:, 0]) + bg2f[...])
    xg2 = jnp.concatenate([_unpack(xg2f[0]), _unpack(xg2f[1])], axis=1)
    x = jnp.concatenate([dx[:, 6:NDF], xg2], axis=1)
    h = x
    for i in range(3):
        h = h + _lrelu(_dot(h, wd[i]) + bd[i])
    x0 = x
    xl = x
    for i in range(2):
        xl = x0 * _dot(xl, cwt[:, i:i + 1]) + cb[i] + xl
    z = jnp.concatenate([h, xl], axis=1)
    p = _lrelu(_dot(z, wp1[...]) + bp1[...])
    y = jax.nn.sigmoid(_dot(p, wp2[...]) + bp2[...])
    y_o[0] = y.reshape(NBF, 8)


def _row_spec(w):
    return pl.BlockSpec((NB, w), lambda i: (i, 0))


def _flat_spec():
    return pl.BlockSpec((NC, 1, NBF, 128), lambda i: (0, i, 0, 0))


def _full_spec(shape):
    nd = len(shape)
    return pl.BlockSpec(shape, lambda i, _n=nd: (0,) * _n)


def _tile_flat(vec32):
    return jnp.stack([jnp.tile(vec32[:F], 8), jnp.tile(vec32[F:], 8)]
                     ).reshape(NC, 1, 128)


def kernel(discrete_x, continous_x, edge_index, edge_attr, churn_date,
           W_g0, b_g0, W_g1, b_g1, W_g2, b_g2, Wd, bd, cw, cb,
           Wp1, bp1, Wp2, bp2):
    f32 = jnp.float32

    ones_tab = jnp.ones((NC * N, F), f32)
    hist = _sc_segsum(ones_tab, edge_index)
    hist_f = hist.reshape(NC, G, NBF, 128)
    hist_spec = pl.BlockSpec((1, 1, NBF, 128), lambda i: (0, i, 0, 0))

    grid = (G,)
    (hs1_f,) = pl.pallas_call(
        _tc_a_body,
        grid=grid,
        in_specs=[_row_spec(NDF), hist_spec,
                  _full_spec((NH, NE)), _full_spec((1, NE)),
                  _full_spec((NE, NE))],
        out_specs=[_flat_spec()],
        out_shape=[jax.ShapeDtypeStruct((NC, G, NBF, 128), f32)],
    )(discrete_x, hist_f, W_g0, b_g0.reshape(1, NE), W_g1)

    s1 = _sc_segsum(hs1_f.reshape(NC * N, F), edge_index)

    eye8 = jnp.eye(8, dtype=f32)
    w2bd = jnp.stack([
        jnp.stack([jnp.kron(eye8, W_g2[16 * h:16 * (h + 1),
                                       16 * co:16 * (co + 1)])
                   for co in range(2)])
        for h in range(2)])

    (hs2_f,) = pl.pallas_call(
        _tc_b_body,
        grid=grid,
        in_specs=[_flat_spec(), _flat_spec(), hist_spec,
                  _full_spec((NC, 1, 128)), _full_spec((2, 2, 128, 128))],
        out_specs=[_flat_spec()],
        out_shape=[jax.ShapeDtypeStruct((NC, G, NBF, 128), f32)],
    )(s1.reshape(NC, G, NBF, 128), hs1_f, hist_f, _tile_flat(b_g1), w2bd)

    s2 = _sc_segsum(hs2_f.reshape(NC * N, F), edge_index)

    (y,) = pl.pallas_call(
        _tc_c_body,
        grid=grid,
        in_specs=[_row_spec(NDF), _flat_spec(), _flat_spec(), hist_spec,
                  _full_spec((NC, 1, 128)),
                  _full_spec((3, D, D)), _full_spec((3, 1, D)),
                  _full_spec((D, 2)), _full_spec((2, 1, D)),
                  _full_spec((2 * D, HID)), _full_spec((1, HID)),
                  _full_spec((HID, 1)), _full_spec((1, 1))],
        out_specs=[pl.BlockSpec((1, NBF, 8), lambda i: (i, 0, 0))],
        out_shape=[jax.ShapeDtypeStruct((G, NBF, 8), f32)],
    )(discrete_x, s2.reshape(NC, G, NBF, 128), hs2_f, hist_f,
      _tile_flat(b_g2), Wd, bd.reshape(3, 1, D), cw.T, cb.reshape(2, 1, D),
      Wp1, bp1.reshape(1, HID), Wp2, bp2.reshape(1, 1))

    return y.reshape(N)


# speedup vs baseline: 35.5864x; 1.0131x over previous
import functools

import jax
import jax.numpy as jnp
from jax import lax
from jax.experimental import pallas as pl
from jax.experimental.pallas import tpu as pltpu
from jax.experimental.pallas import tpu_sc as plsc

N = 100000
E = 1600000
NDF = 38
NH = 32
NE = 32
D = 64
HID = 64

NC = 2
NS = 16
F = 16

N_ACC = N
ROWS_PER_TILE = N_ACC // NS
ZCHUNK = 125
SEG_CHUNK = 800
NCHUNK = (E // NS) // SEG_CHUNK

_MESH = plsc.VectorSubcoreMesh(
    core_axis_name="c", subcore_axis_name="s", num_cores=NC, num_subcores=NS)


def _fill_rows(ref, nrows, vec):
    @pl.loop(0, nrows)
    def _(i):
        ref[i] = vec


def _zero_acc(acc_sh, zeros_v, s):
    base = s * ROWS_PER_TILE

    @pl.loop(0, ROWS_PER_TILE // ZCHUNK)
    def _(j):
        pltpu.sync_copy(zeros_v, acc_sh.at[pl.ds(base + j * ZCHUNK, ZCHUNK)])


@functools.partial(
    pl.kernel,
    out_type=jax.ShapeDtypeStruct((NC * N_ACC, F), jnp.float32),
    mesh=_MESH,
    compiler_params=pltpu.CompilerParams(use_tc_tiling_on_sc=False),
    scratch_types=[
        pltpu.VMEM((SEG_CHUNK,), jnp.int32),
        pltpu.VMEM((SEG_CHUNK,), jnp.int32),
        pltpu.VMEM((SEG_CHUNK,), jnp.int32),
        pltpu.VMEM((SEG_CHUNK,), jnp.int32),
        pltpu.VMEM((SEG_CHUNK, F), jnp.float32),
        pltpu.VMEM((SEG_CHUNK, F), jnp.float32),
        pltpu.VMEM((ZCHUNK, F), jnp.float32),
        pltpu.VMEM_SHARED((N_ACC, F), jnp.float32),
        pltpu.SemaphoreType.DMA,
        pltpu.SemaphoreType.DMA,
        pltpu.SemaphoreType.DMA,
        pltpu.SemaphoreType.DMA,
    ],
)
def _sc_segsum(hs_hbm, ei_hbm, out_hbm,
               src0, src1, dst0, dst1, rows0, rows1, zeros_v, acc_sh,
               sem_i0, sem_i1, sem_g0, sem_g1):
    c = lax.axis_index("c")
    s = lax.axis_index("s")
    _fill_rows(zeros_v, ZCHUNK, jnp.zeros((16,), jnp.float32))
    _zero_acc(acc_sh, zeros_v, s)
    plsc.subcore_barrier()

    edges_per_tile = E // NS
    tile_base = s * edges_per_tile
    half = hs_hbm.at[pl.ds(pl.multiple_of(c * N, 8), N)]
    src_hbm = ei_hbm.at[0]
    dst_hbm = ei_hbm.at[1]
    bufs = ((src0, dst0, rows0, sem_i0, sem_g0),
            (src1, dst1, rows1, sem_i1, sem_g1))

    def idx_start(ck, b):
        s_, d_, _, si, _ = bufs[b]
        off = pl.multiple_of(tile_base + ck * SEG_CHUNK, 8)
        pltpu.async_copy(src_hbm.at[pl.ds(off, SEG_CHUNK)], s_, si)
        pltpu.async_copy(dst_hbm.at[pl.ds(off, SEG_CHUNK)], d_, si)

    def idx_wait(b):
        s_, d_, _, si, _ = bufs[b]
        pltpu.make_async_copy(src_hbm.at[pl.ds(0, SEG_CHUNK)], s_, si).wait()
        pltpu.make_async_copy(dst_hbm.at[pl.ds(0, SEG_CHUNK)], d_, si).wait()

    def gather_start(b):
        s_, _, r_, _, sg = bufs[b]
        pltpu.async_copy(half.at[s_], r_, sg)

    def gather_wait(b):
        s_, _, r_, _, sg = bufs[b]
        pltpu.make_async_copy(half.at[s_], r_, sg).wait()

    def scatter(b):
        _, d_, r_, _, _ = bufs[b]
        pltpu.sync_copy(r_, acc_sh.at[d_], add=True)

    idx_start(0, 0)
    idx_start(1, 1)
    idx_wait(0)
    gather_start(0)

    @pl.loop(0, NCHUNK // 2)
    def _(j):
        for b in (0, 1):
            ck = 2 * j + b
            gather_wait(b)

            @pl.when(ck + 1 < NCHUNK)
            def _():
                idx_wait(1 - b)
                gather_start(1 - b)

            scatter(b)

            @pl.when(ck + 2 < NCHUNK)
            def _():
                idx_start(ck + 2, b)

    if NCHUNK % 2:
        gather_wait(0)
        scatter(0)

    plsc.subcore_barrier()
    row = s * ROWS_PER_TILE
    pltpu.sync_copy(acc_sh.at[pl.ds(row, ROWS_PER_TILE)],
                    out_hbm.at[pl.ds(c * N_ACC + row, ROWS_PER_TILE)])


NB = 5000
NBF = NB // 8
G = N // NB


def _lrelu(x):
    return jnp.maximum(x, 0.01 * x)


def _dot(a, b):
    return jnp.dot(a, b, preferred_element_type=jnp.float32)


def _unpack(flat):
    parts = [flat[:, 16 * k:16 * (k + 1)] for k in range(8)]
    return jnp.stack(parts, axis=1).reshape(NB, F)


def _pack(half):
    h3 = half.reshape(NBF, 8, F)
    return jnp.concatenate([h3[:, k] for k in range(8)], axis=1)


def _dinv_flat(hist):
    return lax.rsqrt(1.0 + hist[0, 0])


def _tc_a_body(dx, wg0, bg0, wg1, h1_o):
    x_d = dx[:, 6:NDF]
    xg0 = _lrelu(_dot(x_d, wg0[...]) + bg0[...])
    h1 = _dot(xg0, wg1[...])
    h1_o[0, 0] = _pack(h1[:, :F])
    h1_o[1, 0] = _pack(h1[:, F:])


def _tc_scale_body(h1_r, hist, hs1_o):
    dinvb = _dinv_flat(hist)
    for c in range(2):
        hs1_o[c, 0] = h1_r[c, 0] * dinvb


def _tc_b_body(s1_r, hs1_r, hist, bg1f, w2bd, hs2_o):
    dinvb = _dinv_flat(hist)
    xg1f = _lrelu(dinvb * (s1_r[:, 0] + hs1_r[:, 0]) + bg1f[...])
    for co in range(2):
        acc = _dot(xg1f[0], w2bd[0, co]) + _dot(xg1f[1], w2bd[1, co])
        hs2_o[co, 0] = acc * dinvb


def _tc_c_body(dx, s2_r, hs2_r, hist, bg2f, wd, bd, cwt, cb,
               wp1, bp1, wp2, bp2, y_o):
    xg2f = _lrelu(_dinv_flat(hist) * (s2_r[:, 0] + hs2_r[:, 0]) + bg2f[...])
    xg2 = jnp.concatenate([_unpack(xg2f[0]), _unpack(xg2f[1])], axis=1)
    x = jnp.concatenate([dx[:, 6:NDF], xg2], axis=1)
    h = x
    for i in range(3):
        h = h + _lrelu(_dot(h, wd[i]) + bd[i])
    x0 = x
    xl = x
    for i in range(2):
        xl = x0 * _dot(xl, cwt[:, i:i + 1]) + cb[i] + xl
    z = jnp.concatenate([h, xl], axis=1)
    p = _lrelu(_dot(z, wp1[...]) + bp1[...])
    y = jax.nn.sigmoid(_dot(p, wp2[...]) + bp2[...])
    y_o[0] = y.reshape(NBF, 8)


def _row_spec(w):
    return pl.BlockSpec((NB, w), lambda i: (i, 0))


def _flat_spec():
    return pl.BlockSpec((NC, 1, NBF, 128), lambda i: (0, i, 0, 0))


def _full_spec(shape):
    nd = len(shape)
    return pl.BlockSpec(shape, lambda i, _n=nd: (0,) * _n)


def _tile_flat(vec32):
    return jnp.stack([jnp.tile(vec32[:F], 8), jnp.tile(vec32[F:], 8)]
                     ).reshape(NC, 1, 128)


def kernel(discrete_x, continous_x, edge_index, edge_attr, churn_date,
           W_g0, b_g0, W_g1, b_g1, W_g2, b_g2, Wd, bd, cw, cb,
           Wp1, bp1, Wp2, bp2):
    f32 = jnp.float32

    ones_tab = jnp.ones((NC * N, F), f32)
    hist = _sc_segsum(ones_tab, edge_index)
    hist_f = hist.reshape(NC, G, NBF, 128)
    hist_spec = pl.BlockSpec((1, 1, NBF, 128), lambda i: (0, i, 0, 0))

    grid = (G,)
    (h1_f,) = pl.pallas_call(
        _tc_a_body,
        grid=grid,
        in_specs=[_row_spec(NDF),
                  _full_spec((NH, NE)), _full_spec((1, NE)),
                  _full_spec((NE, NE))],
        out_specs=[_flat_spec()],
        out_shape=[jax.ShapeDtypeStruct((NC, G, NBF, 128), f32)],
    )(discrete_x, W_g0, b_g0.reshape(1, NE), W_g1)

    (hs1_f,) = pl.pallas_call(
        _tc_scale_body,
        grid=grid,
        in_specs=[_flat_spec(), hist_spec],
        out_specs=[_flat_spec()],
        out_shape=[jax.ShapeDtypeStruct((NC, G, NBF, 128), f32)],
    )(h1_f, hist_f)

    s1 = _sc_segsum(hs1_f.reshape(NC * N, F), edge_index)

    eye8 = jnp.eye(8, dtype=f32)
    w2bd = jnp.stack([
        jnp.stack([jnp.kron(eye8, W_g2[16 * h:16 * (h + 1),
                                       16 * co:16 * (co + 1)])
                   for co in range(2)])
        for h in range(2)])

    (hs2_f,) = pl.pallas_call(
        _tc_b_body,
        grid=grid,
        in_specs=[_flat_spec(), _flat_spec(), hist_spec,
                  _full_spec((NC, 1, 128)), _full_spec((2, 2, 128, 128))],
        out_specs=[_flat_spec()],
        out_shape=[jax.ShapeDtypeStruct((NC, G, NBF, 128), f32)],
    )(s1.reshape(NC, G, NBF, 128), hs1_f, hist_f, _tile_flat(b_g1), w2bd)

    s2 = _sc_segsum(hs2_f.reshape(NC * N, F), edge_index)

    (y,) = pl.pallas_call(
        _tc_c_body,
        grid=grid,
        in_specs=[_row_spec(NDF), _flat_spec(), _flat_spec(), hist_spec,
                  _full_spec((NC, 1, 128)),
                  _full_spec((3, D, D)), _full_spec((3, 1, D)),
                  _full_spec((D, 2)), _full_spec((2, 1, D)),
                  _full_spec((2 * D, HID)), _full_spec((1, HID)),
                  _full_spec((HID, 1)), _full_spec((1, 1))],
        out_specs=[pl.BlockSpec((1, NBF, 8), lambda i: (i, 0, 0))],
        out_shape=[jax.ShapeDtypeStruct((G, NBF, 8), f32)],
    )(discrete_x, s2.reshape(NC, G, NBF, 128), hs2_f, hist_f,
      _tile_flat(b_g2), Wd, bd.reshape(3, 1, D), cw.T, cb.reshape(2, 1, D),
      Wp1, bp1.reshape(1, HID), Wp2, bp2.reshape(1, 1))

    return y.reshape(N)
